# baseline (speedup 1.0000x reference)
import jax
import jax.numpy as jnp
from jax import lax
from jax.experimental import pallas as pl
from jax.experimental.pallas import tpu as pltpu
from jax.experimental.pallas import tpu_sc as plsc

_N_CLS = 1000
_ROWS = 16384
_NC, _NS, _L = 2, 16, 16
_NW = _NC * _NS
_RPW = _ROWS // _NW
_RCH = 16
_NCH = _RPW // _RCH
_SCALE = 1.0 / (_ROWS * _N_CLS)


def _row_cost(buf, r, tf, lane_f, mask_hi):
    base = lane_f - tf
    accs = [jnp.zeros((_L,), jnp.float32) for _ in range(4)]
    for c in range(62):
        p = buf[r, pl.ds(c * _L, _L)]
        accs[c % 4] = accs[c % 4] + jnp.abs(base + jnp.float32(c * 16)) * p
    p = buf[r, pl.ds(984, _L)]
    w = jnp.abs(base + jnp.float32(984)) * mask_hi
    return (accs[0] + accs[1]) + (accs[2] + accs[3]) + w * p


def _consume(buf, t_v, row0, lane_f, mask_hi, acc):
    def body(r, acc):
        tf = t_v[row0 + r, :].astype(jnp.float32)
        return acc + _row_cost(buf, r, tf, lane_f, mask_hi)

    return lax.fori_loop(0, _RCH, body, acc)


def _sc_body(p_hbm, t_hbm, out_hbm, t_v, buf0, buf1, acc_v, sem0, sem1):
    cid = lax.axis_index("c")
    sid = lax.axis_index("s")
    wid = sid * _NC + cid
    base = wid * _RPW

    pltpu.sync_copy(t_hbm.at[pl.ds(base, _RPW)], t_v)
    pltpu.async_copy(p_hbm.at[pl.ds(base, _RCH)], buf0, sem0)
    pltpu.async_copy(p_hbm.at[pl.ds(base + _RCH, _RCH)], buf1, sem1)

    lane = lax.iota(jnp.int32, _L)
    lane_f = lane.astype(jnp.float32)
    mask_hi = jnp.where(lane >= 8, jnp.float32(1.0), jnp.float32(0.0))

    def pair(g2, acc):
        c0 = 2 * g2
        rb0 = base + c0 * _RCH
        pltpu.make_async_copy(p_hbm.at[pl.ds(rb0, _RCH)], buf0, sem0).wait()
        acc = _consume(buf0, t_v, c0 * _RCH, lane_f, mask_hi, acc)

        @pl.when(c0 + 2 < _NCH)
        def _():
            pltpu.async_copy(
                p_hbm.at[pl.ds(base + (c0 + 2) * _RCH, _RCH)], buf0, sem0)

        rb1 = base + (c0 + 1) * _RCH
        pltpu.make_async_copy(p_hbm.at[pl.ds(rb1, _RCH)], buf1, sem1).wait()
        acc = _consume(buf1, t_v, (c0 + 1) * _RCH, lane_f, mask_hi, acc)

        @pl.when(c0 + 3 < _NCH)
        def _():
            pltpu.async_copy(
                p_hbm.at[pl.ds(base + (c0 + 3) * _RCH, _RCH)], buf1, sem1)

        return acc

    acc = lax.fori_loop(0, _NCH // 2, pair, jnp.zeros((_L,), jnp.float32))
    acc_v[...] = acc * jnp.float32(_SCALE)
    pltpu.sync_copy(acc_v, out_hbm.at[wid])


def kernel(output_probs, target_class):
    t32 = target_class.astype(jnp.int32)
    t_mat = jnp.broadcast_to(t32[:, None], (_ROWS, _L))
    mesh = plsc.VectorSubcoreMesh(core_axis_name="c", subcore_axis_name="s")
    f = pl.kernel(
        _sc_body,
        mesh=mesh,
        out_type=jax.ShapeDtypeStruct((_NW, _L), jnp.float32),
        scratch_types=[
            pltpu.VMEM((_RPW, _L), jnp.int32),
            pltpu.VMEM((_RCH, _N_CLS), jnp.float32),
            pltpu.VMEM((_RCH, _N_CLS), jnp.float32),
            pltpu.VMEM((_L,), jnp.float32),
            pltpu.SemaphoreType.DMA,
            pltpu.SemaphoreType.DMA,
        ],
    )
    out = f(output_probs, t_mat)
    return jnp.sum(out)


# speedup vs baseline: 1.1519x; 1.1519x over previous
import jax
import jax.numpy as jnp
from jax import lax
from jax.experimental import pallas as pl
from jax.experimental.pallas import tpu as pltpu
from jax.experimental.pallas import tpu_sc as plsc

_N_CLS = 1000
_ROWS = 16384
_NC, _NS, _L = 2, 16, 16
_NW = _NC * _NS
_SCALE = 1.0 / (_ROWS * _N_CLS)

_SC_ROWS = 6144
_TC_ROWS = _ROWS - _SC_ROWS
_RPW = _SC_ROWS // _NW
_RCH = 16
_NCH = _RPW // _RCH

_BR = 1024
_TC_GRID = _TC_ROWS // _BR


def _row_cost(buf, r, tf, lane_f, mask_hi):
    base = lane_f - tf
    accs = [jnp.zeros((_L,), jnp.float32) for _ in range(4)]
    for c in range(62):
        p = buf[r, pl.ds(c * _L, _L)]
        accs[c % 4] = accs[c % 4] + jnp.abs(base + jnp.float32(c * 16)) * p
    p = buf[r, pl.ds(984, _L)]
    w = jnp.abs(base + jnp.float32(984)) * mask_hi
    return (accs[0] + accs[1]) + (accs[2] + accs[3]) + w * p


def _consume(buf, t_v, row0, lane_f, mask_hi, acc):
    def body(r, acc):
        tf = t_v[row0 + r, :].astype(jnp.float32)
        return acc + _row_cost(buf, r, tf, lane_f, mask_hi)

    return lax.fori_loop(0, _RCH, body, acc)


def _sc_body(p_hbm, t_hbm, out_hbm, t_v, buf0, buf1, acc_v, sem0, sem1):
    cid = lax.axis_index("c")
    sid = lax.axis_index("s")
    wid = sid * _NC + cid
    base = _TC_ROWS + wid * _RPW

    pltpu.sync_copy(t_hbm.at[pl.ds(base, _RPW)], t_v)
    pltpu.async_copy(p_hbm.at[pl.ds(base, _RCH)], buf0, sem0)
    pltpu.async_copy(p_hbm.at[pl.ds(base + _RCH, _RCH)], buf1, sem1)

    lane = lax.iota(jnp.int32, _L)
    lane_f = lane.astype(jnp.float32)
    mask_hi = jnp.where(lane >= 8, jnp.float32(1.0), jnp.float32(0.0))

    def pair(g2, acc):
        c0 = 2 * g2
        rb0 = base + c0 * _RCH
        pltpu.make_async_copy(p_hbm.at[pl.ds(rb0, _RCH)], buf0, sem0).wait()
        acc = _consume(buf0, t_v, c0 * _RCH, lane_f, mask_hi, acc)

        @pl.when(c0 + 2 < _NCH)
        def _():
            pltpu.async_copy(
                p_hbm.at[pl.ds(base + (c0 + 2) * _RCH, _RCH)], buf0, sem0)

        rb1 = base + (c0 + 1) * _RCH
        pltpu.make_async_copy(p_hbm.at[pl.ds(rb1, _RCH)], buf1, sem1).wait()
        acc = _consume(buf1, t_v, (c0 + 1) * _RCH, lane_f, mask_hi, acc)

        @pl.when(c0 + 3 < _NCH)
        def _():
            pltpu.async_copy(
                p_hbm.at[pl.ds(base + (c0 + 3) * _RCH, _RCH)], buf1, sem1)

        return acc

    acc = lax.fori_loop(0, _NCH // 2, pair, jnp.zeros((_L,), jnp.float32))
    acc_v[...] = acc * jnp.float32(_SCALE)
    pltpu.sync_copy(acc_v, out_hbm.at[wid])


def _tc_body(t_ref, p_ref, o_ref):
    i = pl.program_id(0)
    t = t_ref[...]
    j = lax.broadcasted_iota(jnp.int32, (_BR, _N_CLS), 1).astype(jnp.float32)
    w = jnp.abs(j - t) * jnp.float32(_SCALE)
    partial = jnp.sum(w * p_ref[...])

    @pl.when(i == 0)
    def _init():
        o_ref[0, 0] = 0.0

    o_ref[0, 0] += partial


def kernel(output_probs, target_class):
    t32 = target_class.astype(jnp.int32)

    t_mat = jnp.broadcast_to(t32[:, None], (_ROWS, _L))
    mesh = plsc.VectorSubcoreMesh(core_axis_name="c", subcore_axis_name="s")
    sc_fn = pl.kernel(
        _sc_body,
        mesh=mesh,
        out_type=jax.ShapeDtypeStruct((_NW, _L), jnp.float32),
        scratch_types=[
            pltpu.VMEM((_RPW, _L), jnp.int32),
            pltpu.VMEM((_RCH, _N_CLS), jnp.float32),
            pltpu.VMEM((_RCH, _N_CLS), jnp.float32),
            pltpu.VMEM((_L,), jnp.float32),
            pltpu.SemaphoreType.DMA,
            pltpu.SemaphoreType.DMA,
        ],
    )
    sc_out = sc_fn(output_probs, t_mat)

    t_tc = t32.astype(jnp.float32).reshape(_ROWS, 1)
    tc_out = pl.pallas_call(
        _tc_body,
        grid=(_TC_GRID,),
        in_specs=[
            pl.BlockSpec((_BR, 1), lambda i: (i, 0)),
            pl.BlockSpec((_BR, _N_CLS), lambda i: (i, 0)),
        ],
        out_specs=pl.BlockSpec(memory_space=pltpu.SMEM),
        out_shape=jax.ShapeDtypeStruct((1, 1), jnp.float32),
    )(t_tc, output_probs)
    return tc_out[0, 0] + jnp.sum(sc_out)


# speedup vs baseline: 1.1551x; 1.0028x over previous
import jax
import jax.numpy as jnp
from jax import lax
from jax.experimental import pallas as pl
from jax.experimental.pallas import tpu as pltpu
from jax.experimental.pallas import tpu_sc as plsc

_N_CLS = 1000
_ROWS = 16384
_NC, _NS, _L = 2, 16, 16
_NW = _NC * _NS
_SCALE = 1.0 / (_ROWS * _N_CLS)

_SC_ROWS = 6144
_TC_ROWS = _ROWS - _SC_ROWS
_RPW = _SC_ROWS // _NW
_RCH = 16
_NCH = _RPW // _RCH

_BR = 1024
_TC_GRID = _TC_ROWS // _BR


def _row_cost(buf, r, tf, lane_f, mask_hi):
    base = lane_f - tf
    accs = [jnp.zeros((_L,), jnp.float32) for _ in range(4)]
    for c in range(62):
        p = buf[r, pl.ds(c * _L, _L)]
        accs[c % 4] = accs[c % 4] + jnp.abs(base + jnp.float32(c * 16)) * p
    p = buf[r, pl.ds(984, _L)]
    w = jnp.abs(base + jnp.float32(984)) * mask_hi
    return (accs[0] + accs[1]) + (accs[2] + accs[3]) + w * p


def _consume(buf, t_v, row0, lane_f, mask_hi, acc):
    def body(r, acc):
        tf = t_v[row0 + r, :].astype(jnp.float32)
        return acc + _row_cost(buf, r, tf, lane_f, mask_hi)

    return lax.fori_loop(0, _RCH, body, acc)


def _sc_body(p_hbm, t_hbm, out_hbm, t_v, buf0, buf1, acc_v, sem0, sem1):
    cid = lax.axis_index("c")
    sid = lax.axis_index("s")
    wid = sid * _NC + cid
    base = _TC_ROWS + wid * _RPW

    pltpu.sync_copy(t_hbm.at[pl.ds(base, _RPW)], t_v)
    pltpu.async_copy(p_hbm.at[pl.ds(base, _RCH)], buf0, sem0)
    pltpu.async_copy(p_hbm.at[pl.ds(base + _RCH, _RCH)], buf1, sem1)

    lane = lax.iota(jnp.int32, _L)
    lane_f = lane.astype(jnp.float32)
    mask_hi = jnp.where(lane >= 8, jnp.float32(1.0), jnp.float32(0.0))

    def pair(g2, acc):
        c0 = 2 * g2
        rb0 = base + c0 * _RCH
        pltpu.make_async_copy(p_hbm.at[pl.ds(rb0, _RCH)], buf0, sem0).wait()
        acc = _consume(buf0, t_v, c0 * _RCH, lane_f, mask_hi, acc)

        @pl.when(c0 + 2 < _NCH)
        def _():
            pltpu.async_copy(
                p_hbm.at[pl.ds(base + (c0 + 2) * _RCH, _RCH)], buf0, sem0)

        rb1 = base + (c0 + 1) * _RCH
        pltpu.make_async_copy(p_hbm.at[pl.ds(rb1, _RCH)], buf1, sem1).wait()
        acc = _consume(buf1, t_v, (c0 + 1) * _RCH, lane_f, mask_hi, acc)

        @pl.when(c0 + 3 < _NCH)
        def _():
            pltpu.async_copy(
                p_hbm.at[pl.ds(base + (c0 + 3) * _RCH, _RCH)], buf1, sem1)

        return acc

    acc = lax.fori_loop(0, _NCH // 2, pair, jnp.zeros((_L,), jnp.float32))
    acc_v[...] = acc * jnp.float32(_SCALE)
    pltpu.sync_copy(acc_v, out_hbm.at[wid])


def _tc_body(t_ref, p_ref, o_ref):
    i = pl.program_id(0)
    t = t_ref[...]
    j = lax.broadcasted_iota(jnp.int32, (_BR, _N_CLS), 1).astype(jnp.float32)
    w = jnp.abs(j - t) * jnp.float32(_SCALE)
    partial = jnp.sum(w * p_ref[...])

    @pl.when(i == 0)
    def _init():
        o_ref[0, 0] = 0.0

    o_ref[0, 0] += partial


def kernel(output_probs, target_class):
    t32 = target_class.astype(jnp.int32)

    t_mat = jnp.broadcast_to(t32[:, None], (_ROWS, _L))
    mesh = plsc.VectorSubcoreMesh(core_axis_name="c", subcore_axis_name="s")
    sc_fn = pl.kernel(
        _sc_body,
        mesh=mesh,
        out_type=jax.ShapeDtypeStruct((_NW, _L), jnp.float32),
        scratch_types=[
            pltpu.VMEM((_RPW, _L), jnp.int32),
            pltpu.VMEM((_RCH, _N_CLS), jnp.float32),
            pltpu.VMEM((_RCH, _N_CLS), jnp.float32),
            pltpu.VMEM((_L,), jnp.float32),
            pltpu.SemaphoreType.DMA,
            pltpu.SemaphoreType.DMA,
        ],
    )
    sc_out = sc_fn(output_probs, t_mat)

    t_tc = t32.astype(jnp.float32).reshape(_ROWS, 1)
    tc_out = pl.pallas_call(
        _tc_body,
        grid=(_TC_GRID,),
        in_specs=[
            pl.BlockSpec((_BR, 1), lambda i: (i, 0)),
            pl.BlockSpec((_BR, _N_CLS), lambda i: (i, 0)),
        ],
        out_specs=pl.BlockSpec(memory_space=pltpu.SMEM),
        out_shape=jax.ShapeDtypeStruct((1, 1), jnp.float32),
        compiler_params=pltpu.CompilerParams(skip_device_barrier=True),
    )(t_tc, output_probs)
    return tc_out[0, 0] + jnp.sum(sc_out)


# speedup vs baseline: 1.5006x; 1.2991x over previous
import jax
import jax.numpy as jnp
from jax import lax
from jax.experimental import pallas as pl
from jax.experimental.pallas import tpu as pltpu

_N_CLS = 1000
_ROWS = 16384
_SCALE = 1.0 / (_ROWS * _N_CLS)

_BJ = 8
_GRID = _N_CLS // _BJ


def _tc_body(t_ref, p_ref, o_ref):
    i = pl.program_id(0)
    t = t_ref[...]
    j = lax.broadcasted_iota(jnp.int32, (_BJ, _ROWS), 0) + i * _BJ
    w = jnp.abs(j.astype(jnp.float32) - t) * jnp.float32(_SCALE)
    partial = jnp.sum(w * p_ref[...])

    @pl.when(i == 0)
    def _init():
        o_ref[0, 0] = 0.0

    o_ref[0, 0] += partial


def kernel(output_probs, target_class):
    pt = output_probs.T
    t_row = target_class.astype(jnp.float32).reshape(1, _ROWS)
    out = pl.pallas_call(
        _tc_body,
        grid=(_GRID,),
        in_specs=[
            pl.BlockSpec((1, _ROWS), lambda i: (0, 0)),
            pl.BlockSpec((_BJ, _ROWS), lambda i: (i, 0)),
        ],
        out_specs=pl.BlockSpec(memory_space=pltpu.SMEM),
        out_shape=jax.ShapeDtypeStruct((1, 1), jnp.float32),
    )(t_row, pt)
    return out[0, 0]


# speedup vs baseline: 3.8845x; 2.5886x over previous
import jax
import jax.numpy as jnp
from jax import lax
from jax.experimental import pallas as pl
from jax.experimental.pallas import tpu as pltpu

_N_CLS = 1000
_ROWS = 16384
_SCALE = 1.0 / (_ROWS * _N_CLS)

_BJ = 8
_NSTREAM = 5
_GRID = _N_CLS // (_BJ * _NSTREAM)


def _tc_body(t_ref, *rest):
    p_refs, o_ref = rest[:_NSTREAM], rest[_NSTREAM]
    i = pl.program_id(0)
    t = t_ref[...]
    partial = jnp.float32(0.0)
    for s, p_ref in enumerate(p_refs):
        base = (_NSTREAM * i + s) * _BJ
        j = lax.broadcasted_iota(jnp.int32, (_BJ, _ROWS), 0) + base
        w = jnp.abs(j.astype(jnp.float32) - t) * jnp.float32(_SCALE)
        partial += jnp.sum(w * p_ref[...])

    @pl.when(i == 0)
    def _init():
        o_ref[0, 0] = 0.0

    o_ref[0, 0] += partial


def kernel(output_probs, target_class):
    pt = output_probs.T
    t_row = target_class.astype(jnp.float32).reshape(1, _ROWS)
    in_specs = [pl.BlockSpec((1, _ROWS), lambda i: (0, 0))]
    for s in range(_NSTREAM):
        in_specs.append(
            pl.BlockSpec((_BJ, _ROWS), lambda i, s=s: (_NSTREAM * i + s, 0)))
    out = pl.pallas_call(
        _tc_body,
        grid=(_GRID,),
        in_specs=in_specs,
        out_specs=pl.BlockSpec(memory_space=pltpu.SMEM),
        out_shape=jax.ShapeDtypeStruct((1, 1), jnp.float32),
    )(t_row, *([pt] * _NSTREAM))
    return out[0, 0]


# speedup vs baseline: 5.3830x; 1.3858x over previous
import jax
import jax.numpy as jnp
from jax import lax
from jax.experimental import pallas as pl
from jax.experimental.pallas import tpu as pltpu

_N_CLS = 1000
_ROWS = 16384
_SCALE = 1.0 / (_ROWS * _N_CLS)

_BJ = 8
_NSTREAM = 25
_GRID = _N_CLS // (_BJ * _NSTREAM)


def _tc_body(t_ref, *rest):
    p_refs, o_ref = rest[:_NSTREAM], rest[_NSTREAM]
    i = pl.program_id(0)
    t = t_ref[...]
    partial = jnp.float32(0.0)
    for s, p_ref in enumerate(p_refs):
        base = (_NSTREAM * i + s) * _BJ
        j = lax.broadcasted_iota(jnp.int32, (_BJ, _ROWS), 0) + base
        w = jnp.abs(j.astype(jnp.float32) - t) * jnp.float32(_SCALE)
        partial += jnp.sum(w * p_ref[...])

    @pl.when(i == 0)
    def _init():
        o_ref[0, 0] = 0.0

    o_ref[0, 0] += partial


def kernel(output_probs, target_class):
    pt = output_probs.T
    t_row = target_class.astype(jnp.float32).reshape(1, _ROWS)
    in_specs = [pl.BlockSpec((1, _ROWS), lambda i: (0, 0))]
    for s in range(_NSTREAM):
        in_specs.append(
            pl.BlockSpec((_BJ, _ROWS), lambda i, s=s: (_NSTREAM * i + s, 0)))
    out = pl.pallas_call(
        _tc_body,
        grid=(_GRID,),
        in_specs=in_specs,
        out_specs=pl.BlockSpec(memory_space=pltpu.SMEM),
        out_shape=jax.ShapeDtypeStruct((1, 1), jnp.float32),
    )(t_row, *([pt] * _NSTREAM))
    return out[0, 0]
